# baseline (speedup 1.0000x reference)
import jax
import jax.numpy as jnp
from jax import lax
from jax.experimental import pallas as pl
from jax.experimental.pallas import tpu as pltpu
from jax.experimental.pallas import tpu_sc as plsc

N_STIMULI = 30
N_DIM = 10
BATCH = 16384
N_IDX = N_STIMULI + 1
LUT_PAD = 1024

_INFO = plsc.get_sparse_core_info()
_NC = 1
_NS = _INFO.num_subcores
_NW = _NC * _NS
_L = _INFO.num_lanes
_B_PER_W = BATCH // _NW
_GROUPS = _B_PER_W // _L
_LUT_PER_S = LUT_PAD // _NS


def _newton_sqrt(x):
    i = lax.bitcast_convert_type(x, jnp.int32)
    i = jnp.int32(0x5F3759DF) - lax.shift_right_logical(i, 1)
    y = lax.bitcast_convert_type(i, jnp.float32)
    for _ in range(2):
        y = y * (1.5 - 0.5 * x * y * y)
    return x * y


def _pair_value(table_v, ia, ib):
    fa = ia * N_DIM
    fb = ib * N_DIM
    acc = jnp.full((_L,), 1e-12, jnp.float32)
    for d in range(N_DIM):
        av = plsc.load_gather(table_v, [fa + d])
        bv = plsc.load_gather(table_v, [fb + d])
        df = av - bv
        acc = acc + df * df
    dist = _newton_sqrt(acc)
    s = jnp.exp(-3.0 * dist)
    return 1.0 / (1.0 + jnp.exp(-s))


def _sc_body(pid_hbm, table_hbm, out_hbm, pid_v, table_v, lutloc_v, lut_v,
             res_v, lut_sh, sem_t, sem_i, sem_o):
    cid = lax.axis_index("c")
    sid = lax.axis_index("s")
    wid = sid * _NC + cid
    base = wid * _B_PER_W
    ct = pltpu.async_copy(table_hbm, table_v, sem_t)
    c0 = pltpu.async_copy(pid_hbm.at[pl.ds(base, _B_PER_W)], pid_v, sem_i)
    ct.wait()

    iota = lax.iota(jnp.int32, _L)
    for g in range(_LUT_PER_S // _L):
        p = sid * _LUT_PER_S + g * _L + iota
        ia = jnp.minimum(p // N_IDX, N_IDX - 1)
        ib = p % N_IDX
        lutloc_v[pl.ds(g * _L, _L)] = _pair_value(table_v, ia, ib)

    pltpu.sync_copy(lutloc_v, lut_sh.at[pl.ds(sid * _LUT_PER_S, _LUT_PER_S)])
    plsc.subcore_barrier()
    pltpu.sync_copy(lut_sh.at[pl.ds(0, 968)], lut_v.at[pl.ds(0, 968)])
    c0.wait()

    quart = _B_PER_W // 4
    qg = _GROUPS // 4
    copies = []
    for q in range(4):
        for g in range(q * qg, (q + 1) * qg):
            pid = pid_v[pl.ds(g * _L, _L)]
            res_v[pl.ds(g * _L, _L)] = plsc.load_gather(lut_v, [pid])
        copies.append(
            pltpu.async_copy(res_v.at[pl.ds(q * quart, quart)],
                             out_hbm.at[pl.ds(base + q * quart, quart)],
                             sem_o))
    for c in copies:
        c.wait()


@jax.jit
def _run(pid, table):
    mesh = plsc.VectorSubcoreMesh(core_axis_name="c", subcore_axis_name="s",
                                  num_cores=_NC)
    fn = pl.kernel(
        _sc_body,
        mesh=mesh,
        out_type=jax.ShapeDtypeStruct((BATCH,), jnp.float32),
        compiler_params=pltpu.CompilerParams(needs_layout_passes=False),
        scratch_types=[
            pltpu.VMEM((_B_PER_W,), jnp.int32),
            pltpu.VMEM((N_IDX * N_DIM,), jnp.float32),
            pltpu.VMEM((_LUT_PER_S,), jnp.float32),
            pltpu.VMEM((LUT_PAD,), jnp.float32),
            pltpu.VMEM((_B_PER_W,), jnp.float32),
            pltpu.VMEM_SHARED((LUT_PAD,), jnp.float32),
            pltpu.SemaphoreType.DMA,
            pltpu.SemaphoreType.DMA,
            pltpu.SemaphoreType.DMA,
        ],
    )
    return fn(pid, table)


def kernel(rate2_stimulus_set, percept_embeddings):
    idx = rate2_stimulus_set.astype(jnp.int32)
    pid = idx[:, 0] * N_IDX + idx[:, 1]
    out = _run(pid, percept_embeddings.reshape(-1))
    return out.reshape(BATCH, 1)
